# Initial kernel scaffold; baseline (speedup 1.0000x reference)
import jax
import jax.numpy as jnp
from jax.experimental import pallas as pl


def kernel(x, B_indices, B_values, bc_value, interior_flag, W):
    raise NotImplementedError("write your pallas kernel here")


# speedup vs baseline: 6.2392x; 6.2392x over previous
import functools

import jax
import jax.numpy as jnp
from jax import lax
from jax.experimental import pallas as pl
from jax.experimental.pallas import tpu as pltpu
from jax.experimental.pallas import tpu_sc as plsc

N = 4096
BATCH = 64
NW = 32
ROWS_PER_W = N // NW
K = 512
GSUB = 128
NQ = BATCH // 16


def _sc_spmv(xT, cols, rows_arr, vals, starts):
    mesh = plsc.VectorSubcoreMesh(core_axis_name="c", subcore_axis_name="s")

    @functools.partial(
        pl.kernel,
        out_type=jax.ShapeDtypeStruct((N, BATCH), jnp.float32),
        mesh=mesh,
        scratch_types=[
            pltpu.VMEM((K,), jnp.int32),
            pltpu.VMEM((K,), jnp.int32),
            pltpu.VMEM((K,), jnp.float32),
            pltpu.VMEM((K, BATCH), jnp.float32),
            pltpu.VMEM((ROWS_PER_W, BATCH), jnp.float32),
            pltpu.VMEM((64,), jnp.int32),
            pltpu.SemaphoreType.DMA,
        ],
        compiler_params=pltpu.CompilerParams(use_tc_tiling_on_sc=False),
    )
    def k(xT_hbm, cols_hbm, rows_hbm, vals_hbm, starts_hbm,
          out_hbm, colv, rowv, valv, gath, acc, startsv, sem):
        wid = lax.axis_index("s") * 2 + lax.axis_index("c")
        row_base = wid * ROWS_PER_W

        pltpu.sync_copy(starts_hbm, startsv)
        svec = startsv[pl.ds(wid, 16)]
        s0 = svec[0]
        s1 = svec[1]
        base0 = (s0 // 8) * 8

        def zbody(r, _):
            for q in range(NQ):
                acc[r, pl.ds(q * 16, 16)] = jnp.zeros((16,), jnp.float32)
            return 0
        lax.fori_loop(0, ROWS_PER_W, zbody, 0)

        nchunks = (s1 - base0 + (K - 1)) // K

        def chunk_body(c, _):
            base = pl.multiple_of(base0 + c * K, 8)
            pltpu.sync_copy(cols_hbm.at[pl.ds(base, K)], colv)
            pltpu.sync_copy(rows_hbm.at[pl.ds(base, K)], rowv)
            pltpu.sync_copy(vals_hbm.at[pl.ds(base, K)], valv)
            copies = []
            for g in range(K // GSUB):
                copies.append(pltpu.async_copy(
                    xT_hbm.at[colv.at[pl.ds(g * GSUB, GSUB)]],
                    gath.at[pl.ds(g * GSUB, GSUB)], sem))
            for cp in copies:
                cp.wait()

            jlo = jnp.maximum(s0 - base, 0)
            jhi = jnp.minimum(s1 - base, K)

            def gbody(g, _):
                jb = g * 16
                jidx = jb + lax.iota(jnp.int32, 16)
                inr = (jidx >= jlo) & (jidx < jhi)
                rows16 = jnp.clip(rowv[pl.ds(jb, 16)] - row_base,
                                  0, ROWS_PER_W - 1)
                vals16 = jnp.where(inr, valv[pl.ds(jb, 16)],
                                   jnp.zeros((16,), jnp.float32))
                for t in range(16):
                    r = rows16[t]
                    v = vals16[t]
                    for q in range(NQ):
                        sl = pl.ds(q * 16, 16)
                        plsc.addupdate(acc.at[r, sl], v * gath[jb + t, sl])
                return 0
            lax.fori_loop(jlo // 16, (jhi + 15) // 16, gbody, 0)
            return 0
        lax.fori_loop(0, nchunks, chunk_body, 0)

        pltpu.sync_copy(acc, out_hbm.at[pl.ds(row_base, ROWS_PER_W)])

    return k(xT, cols, rows_arr, vals, starts)


def _tc_linear(p, W, bc, flag):
    JBLK = 512

    def body(p_ref, bci_ref, fi_ref, w_ref, bcj_ref, fj_ref, o_ref):
        h2 = bci_ref[...] + p_ref[...] * fi_ref[...]
        o = lax.dot_general(h2, w_ref[...],
                            (((0,), (1,)), ((), ())),
                            preferred_element_type=jnp.float32)
        o_ref[...] = bcj_ref[...] + o * fj_ref[...]

    return pl.pallas_call(
        body,
        grid=(N // JBLK,),
        in_specs=[
            pl.BlockSpec((N, BATCH), lambda j: (0, 0)),
            pl.BlockSpec((N, 1), lambda j: (0, 0)),
            pl.BlockSpec((N, 1), lambda j: (0, 0)),
            pl.BlockSpec((JBLK, N), lambda j: (j, 0)),
            pl.BlockSpec((1, JBLK), lambda j: (0, j)),
            pl.BlockSpec((1, JBLK), lambda j: (0, j)),
        ],
        out_specs=pl.BlockSpec((BATCH, JBLK), lambda j: (0, j)),
        out_shape=jax.ShapeDtypeStruct((BATCH, N), jnp.float32),
    )(p, bc.reshape(N, 1), flag.reshape(N, 1), W,
      bc.reshape(1, N), flag.reshape(1, N))


def kernel(x, B_indices, B_values, bc_value, interior_flag, W):
    rows = B_indices[0]
    cols = B_indices[1]
    nnz = B_values.shape[0]
    nnz_pad = ((nnz + K + 7) // 8) * 8 + 8
    pad = nnz_pad - nnz
    cols_p = jnp.pad(cols, (0, pad))
    rows_p = jnp.pad(rows, (0, pad))
    vals_p = jnp.pad(B_values, (0, pad))
    bounds = jnp.arange(0, NW + 1, dtype=jnp.int32) * ROWS_PER_W
    starts = jnp.searchsorted(rows, bounds, side="left").astype(jnp.int32)
    starts = jnp.pad(starts, (0, 64 - (NW + 1)))
    xT = x.T
    p = _sc_spmv(xT, cols_p, rows_p, vals_p, starts)
    return _tc_linear(p, W, bc_value, interior_flag)


# speedup vs baseline: 7.0716x; 1.1334x over previous
import functools

import jax
import jax.numpy as jnp
from jax import lax
from jax.experimental import pallas as pl
from jax.experimental.pallas import tpu as pltpu
from jax.experimental.pallas import tpu_sc as plsc

N = 4096
BATCH = 64
NW = 32
ROWS_PER_W = N // NW
K = 512
GSUB = 128
NQ = BATCH // 16


def _sc_spmv(xT, cols, rows_arr, vals, starts, nnz_pad):
    mesh = plsc.VectorSubcoreMesh(core_axis_name="c", subcore_axis_name="s")
    maxbase = nnz_pad - K

    @functools.partial(
        pl.kernel,
        out_type=jax.ShapeDtypeStruct((N, BATCH), jnp.float32),
        mesh=mesh,
        scratch_types=[
            pltpu.VMEM((K,), jnp.int32),
            pltpu.VMEM((K,), jnp.int32),
            pltpu.VMEM((K,), jnp.int32),
            pltpu.VMEM((K,), jnp.int32),
            pltpu.VMEM((K,), jnp.float32),
            pltpu.VMEM((K,), jnp.float32),
            pltpu.VMEM((K, BATCH), jnp.float32),
            pltpu.VMEM((K, BATCH), jnp.float32),
            pltpu.VMEM((ROWS_PER_W, BATCH), jnp.float32),
            pltpu.VMEM((64,), jnp.int32),
            pltpu.SemaphoreType.DMA,
            pltpu.SemaphoreType.DMA,
            pltpu.SemaphoreType.DMA,
            pltpu.SemaphoreType.DMA,
        ],
        compiler_params=pltpu.CompilerParams(use_tc_tiling_on_sc=False),
    )
    def k(xT_hbm, cols_hbm, rows_hbm, vals_hbm, starts_hbm, out_hbm,
          colv0, colv1, rowv0, rowv1, valv0, valv1, gath0, gath1,
          acc, startsv, semL0, semL1, semG0, semG1):
        wid = lax.axis_index("s") * 2 + lax.axis_index("c")
        row_base = wid * ROWS_PER_W

        pltpu.sync_copy(starts_hbm, startsv)
        svec = startsv[pl.ds(wid, 16)]
        s0 = svec[0]
        s1 = svec[1]
        base0 = (s0 // 8) * 8

        def cbase(c):
            return pl.multiple_of(jnp.minimum(base0 + c * K, maxbase), 8)

        def issue_lin(c, colv, rowv, valv, semL):
            b = cbase(c)
            h = [pltpu.async_copy(cols_hbm.at[pl.ds(b, K)], colv, semL),
                 pltpu.async_copy(rows_hbm.at[pl.ds(b, K)], rowv, semL),
                 pltpu.async_copy(vals_hbm.at[pl.ds(b, K)], valv, semL)]
            return h

        def wait_lin(colv, rowv, valv, semL):
            pltpu.make_async_copy(cols_hbm.at[pl.ds(0, K)], colv, semL).wait()
            pltpu.make_async_copy(rows_hbm.at[pl.ds(0, K)], rowv, semL).wait()
            pltpu.make_async_copy(vals_hbm.at[pl.ds(0, K)], valv, semL).wait()

        def issue_gath(colv, gath, semG):
            for g in range(K // GSUB):
                pltpu.async_copy(
                    xT_hbm.at[colv.at[pl.ds(g * GSUB, GSUB)]],
                    gath.at[pl.ds(g * GSUB, GSUB)], semG)

        def wait_gath(colv, gath, semG):
            for g in range(K // GSUB):
                pltpu.make_async_copy(
                    xT_hbm.at[colv.at[pl.ds(g * GSUB, GSUB)]],
                    gath.at[pl.ds(g * GSUB, GSUB)], semG).wait()

        def compute(c, rowv, valv, gath):
            b = cbase(c)
            jlo = jnp.maximum(s0 - b, 0)
            jhi = jnp.minimum(s1 - b, K)

            def gbody(g, _):
                jb = g * 16
                jidx = jb + lax.iota(jnp.int32, 16)
                inr = (jidx >= jlo) & (jidx < jhi)
                rows16 = jnp.clip(rowv[pl.ds(jb, 16)] - row_base,
                                  0, ROWS_PER_W - 1)
                vals16 = jnp.where(inr, valv[pl.ds(jb, 16)],
                                   jnp.zeros((16,), jnp.float32))
                for t in range(16):
                    r = rows16[t]
                    v = vals16[t]
                    for q in range(NQ):
                        sl = pl.ds(q * 16, 16)
                        plsc.addupdate(acc.at[r, sl], v * gath[jb + t, sl])
                return 0
            lax.fori_loop(jlo // 16, (jhi + 15) // 16, gbody, 0)

        issue_lin(0, colv0, rowv0, valv0, semL0)
        issue_lin(1, colv1, rowv1, valv1, semL1)

        def zbody(r, _):
            for q in range(NQ):
                acc[r, pl.ds(q * 16, 16)] = jnp.zeros((16,), jnp.float32)
            return 0
        lax.fori_loop(0, ROWS_PER_W, zbody, 0)

        wait_lin(colv0, rowv0, valv0, semL0)
        issue_gath(colv0, gath0, semG0)

        nchunks = (s1 - base0 + (K - 1)) // K
        npairs = (nchunks + 1) // 2

        def pair_body(i, _):
            a = 2 * i
            wait_gath(colv0, gath0, semG0)
            wait_lin(colv1, rowv1, valv1, semL1)
            issue_gath(colv1, gath1, semG1)
            compute(a, rowv0, valv0, gath0)
            issue_lin(a + 2, colv0, rowv0, valv0, semL0)
            wait_lin(colv0, rowv0, valv0, semL0)
            issue_gath(colv0, gath0, semG0)
            wait_gath(colv1, gath1, semG1)
            compute(a + 1, rowv1, valv1, gath1)
            issue_lin(a + 3, colv1, rowv1, valv1, semL1)
            return 0
        lax.fori_loop(0, npairs, pair_body, 0)

        wait_gath(colv0, gath0, semG0)
        wait_lin(colv1, rowv1, valv1, semL1)

        pltpu.sync_copy(acc, out_hbm.at[pl.ds(row_base, ROWS_PER_W)])

    return k(xT, cols, rows_arr, vals, starts)


def _tc_linear(p, W, bc, flag):
    JBLK = 512

    def body(p_ref, bci_ref, fi_ref, w_ref, bcj_ref, fj_ref, o_ref):
        h2 = bci_ref[...] + p_ref[...] * fi_ref[...]
        o = lax.dot_general(h2, w_ref[...],
                            (((0,), (1,)), ((), ())),
                            preferred_element_type=jnp.float32)
        o_ref[...] = bcj_ref[...] + o * fj_ref[...]

    return pl.pallas_call(
        body,
        grid=(N // JBLK,),
        in_specs=[
            pl.BlockSpec((N, BATCH), lambda j: (0, 0)),
            pl.BlockSpec((N, 1), lambda j: (0, 0)),
            pl.BlockSpec((N, 1), lambda j: (0, 0)),
            pl.BlockSpec((JBLK, N), lambda j: (j, 0)),
            pl.BlockSpec((1, JBLK), lambda j: (0, j)),
            pl.BlockSpec((1, JBLK), lambda j: (0, j)),
        ],
        out_specs=pl.BlockSpec((BATCH, JBLK), lambda j: (0, j)),
        out_shape=jax.ShapeDtypeStruct((BATCH, N), jnp.float32),
    )(p, bc.reshape(N, 1), flag.reshape(N, 1), W,
      bc.reshape(1, N), flag.reshape(1, N))


def kernel(x, B_indices, B_values, bc_value, interior_flag, W):
    rows = B_indices[0]
    cols = B_indices[1]
    nnz = B_values.shape[0]
    nnz_pad = ((nnz + 2 * K + 7) // 8) * 8 + 8
    pad = nnz_pad - nnz
    cols_p = jnp.pad(cols, (0, pad))
    rows_p = jnp.pad(rows, (0, pad))
    vals_p = jnp.pad(B_values, (0, pad))
    bounds = jnp.arange(0, NW + 1, dtype=jnp.int32) * ROWS_PER_W
    starts = jnp.searchsorted(rows, bounds, side="left").astype(jnp.int32)
    starts = jnp.pad(starts, (0, 64 - (NW + 1)))
    xT = x.T
    p = _sc_spmv(xT, cols_p, rows_p, vals_p, starts, nnz_pad)
    return _tc_linear(p, W, bc_value, interior_flag)


# speedup vs baseline: 10.3230x; 1.4598x over previous
import functools

import jax
import jax.numpy as jnp
from jax import lax
from jax.experimental import pallas as pl
from jax.experimental.pallas import tpu as pltpu
from jax.experimental.pallas import tpu_sc as plsc

N = 4096
BATCH = 64
NW = 32
ROWS_PER_W = N // NW
K = 512
GSUB = 128
NQ = BATCH // 16


def _sc_spmv(xT, cols, rows_arr, vals, starts, nnz_pad):
    mesh = plsc.VectorSubcoreMesh(core_axis_name="c", subcore_axis_name="s")
    maxbase = nnz_pad - K

    @functools.partial(
        pl.kernel,
        out_type=jax.ShapeDtypeStruct((N, BATCH), jnp.float32),
        mesh=mesh,
        scratch_types=[
            pltpu.VMEM((K,), jnp.int32),
            pltpu.VMEM((K,), jnp.int32),
            pltpu.VMEM((K,), jnp.int32),
            pltpu.VMEM((K,), jnp.int32),
            pltpu.VMEM((K,), jnp.float32),
            pltpu.VMEM((K,), jnp.float32),
            pltpu.VMEM((K, BATCH), jnp.float32),
            pltpu.VMEM((K, BATCH), jnp.float32),
            pltpu.VMEM((ROWS_PER_W, BATCH), jnp.float32),
            pltpu.VMEM((64,), jnp.int32),
            pltpu.SemaphoreType.DMA,
            pltpu.SemaphoreType.DMA,
            pltpu.SemaphoreType.DMA,
            pltpu.SemaphoreType.DMA,
        ],
        compiler_params=pltpu.CompilerParams(use_tc_tiling_on_sc=False),
    )
    def k(xT_hbm, cols_hbm, rows_hbm, vals_hbm, starts_hbm, out_hbm,
          colv0, colv1, rowv0, rowv1, valv0, valv1, gath0, gath1,
          acc, startsv, semL0, semL1, semG0, semG1):
        wid = lax.axis_index("s") * 2 + lax.axis_index("c")
        row_base = wid * ROWS_PER_W

        pltpu.sync_copy(starts_hbm, startsv)
        svec = startsv[pl.ds(wid, 16)]
        s0 = svec[0]
        s1 = svec[1]
        base0 = (s0 // 8) * 8

        def cbase(c):
            return pl.multiple_of(jnp.minimum(base0 + c * K, maxbase), 8)

        def issue_lin(c, colv, rowv, valv, semL):
            b = cbase(c)
            h = [pltpu.async_copy(cols_hbm.at[pl.ds(b, K)], colv, semL),
                 pltpu.async_copy(rows_hbm.at[pl.ds(b, K)], rowv, semL),
                 pltpu.async_copy(vals_hbm.at[pl.ds(b, K)], valv, semL)]
            return h

        def wait_lin(colv, rowv, valv, semL):
            pltpu.make_async_copy(cols_hbm.at[pl.ds(0, K)], colv, semL).wait()
            pltpu.make_async_copy(rows_hbm.at[pl.ds(0, K)], rowv, semL).wait()
            pltpu.make_async_copy(vals_hbm.at[pl.ds(0, K)], valv, semL).wait()

        def issue_gath(colv, gath, semG):
            for g in range(K // GSUB):
                pltpu.async_copy(
                    xT_hbm.at[colv.at[pl.ds(g * GSUB, GSUB)]],
                    gath.at[pl.ds(g * GSUB, GSUB)], semG)

        def wait_gath(colv, gath, semG):
            for g in range(K // GSUB):
                pltpu.make_async_copy(
                    xT_hbm.at[colv.at[pl.ds(g * GSUB, GSUB)]],
                    gath.at[pl.ds(g * GSUB, GSUB)], semG).wait()

        def compute(c, rowv, valv, gath):
            b = cbase(c)
            jlo = jnp.maximum(s0 - b, 0)
            jhi = jnp.minimum(s1 - b, K)

            @plsc.parallel_loop(jlo // 16, (jhi + 15) // 16, unroll=2)
            def gbody(g):
                jb = g * 16
                jidx = jb + lax.iota(jnp.int32, 16)
                inr = (jidx >= jlo) & (jidx < jhi)
                rows16 = jnp.clip(rowv[pl.ds(jb, 16)] - row_base,
                                  0, ROWS_PER_W - 1)
                vals16 = jnp.where(inr, valv[pl.ds(jb, 16)],
                                   jnp.zeros((16,), jnp.float32))
                for t in range(16):
                    r = rows16[t]
                    v = vals16[t]
                    for q in range(NQ):
                        sl = pl.ds(q * 16, 16)
                        plsc.addupdate(acc.at[r, sl], v * gath[jb + t, sl])

        issue_lin(0, colv0, rowv0, valv0, semL0)
        issue_lin(1, colv1, rowv1, valv1, semL1)

        def zbody(r, _):
            for q in range(NQ):
                acc[r, pl.ds(q * 16, 16)] = jnp.zeros((16,), jnp.float32)
            return 0
        lax.fori_loop(0, ROWS_PER_W, zbody, 0)

        wait_lin(colv0, rowv0, valv0, semL0)
        issue_gath(colv0, gath0, semG0)

        nchunks = (s1 - base0 + (K - 1)) // K
        npairs = (nchunks + 1) // 2

        def pair_body(i, _):
            a = 2 * i
            wait_gath(colv0, gath0, semG0)
            wait_lin(colv1, rowv1, valv1, semL1)
            issue_gath(colv1, gath1, semG1)
            compute(a, rowv0, valv0, gath0)
            issue_lin(a + 2, colv0, rowv0, valv0, semL0)
            wait_lin(colv0, rowv0, valv0, semL0)
            issue_gath(colv0, gath0, semG0)
            wait_gath(colv1, gath1, semG1)
            compute(a + 1, rowv1, valv1, gath1)
            issue_lin(a + 3, colv1, rowv1, valv1, semL1)
            return 0
        lax.fori_loop(0, npairs, pair_body, 0)

        wait_gath(colv0, gath0, semG0)
        wait_lin(colv1, rowv1, valv1, semL1)

        pltpu.sync_copy(acc, out_hbm.at[pl.ds(row_base, ROWS_PER_W)])

    return k(xT, cols, rows_arr, vals, starts)


def _tc_linear(p, W, bc, flag):
    JBLK = 512

    def body(p_ref, bci_ref, fi_ref, w_ref, bcj_ref, fj_ref, o_ref):
        h2 = bci_ref[...] + p_ref[...] * fi_ref[...]
        o = lax.dot_general(h2, w_ref[...],
                            (((0,), (1,)), ((), ())),
                            preferred_element_type=jnp.float32)
        o_ref[...] = bcj_ref[...] + o * fj_ref[...]

    return pl.pallas_call(
        body,
        grid=(N // JBLK,),
        in_specs=[
            pl.BlockSpec((N, BATCH), lambda j: (0, 0)),
            pl.BlockSpec((N, 1), lambda j: (0, 0)),
            pl.BlockSpec((N, 1), lambda j: (0, 0)),
            pl.BlockSpec((JBLK, N), lambda j: (j, 0)),
            pl.BlockSpec((1, JBLK), lambda j: (0, j)),
            pl.BlockSpec((1, JBLK), lambda j: (0, j)),
        ],
        out_specs=pl.BlockSpec((BATCH, JBLK), lambda j: (0, j)),
        out_shape=jax.ShapeDtypeStruct((BATCH, N), jnp.float32),
    )(p, bc.reshape(N, 1), flag.reshape(N, 1), W,
      bc.reshape(1, N), flag.reshape(1, N))


def kernel(x, B_indices, B_values, bc_value, interior_flag, W):
    rows = B_indices[0]
    cols = B_indices[1]
    nnz = B_values.shape[0]
    nnz_pad = ((nnz + 2 * K + 7) // 8) * 8 + 8
    pad = nnz_pad - nnz
    cols_p = jnp.pad(cols, (0, pad))
    rows_p = jnp.pad(rows, (0, pad))
    vals_p = jnp.pad(B_values, (0, pad))
    bounds = jnp.arange(0, NW + 1, dtype=jnp.int32) * ROWS_PER_W
    starts = jnp.searchsorted(rows, bounds, side="left").astype(jnp.int32)
    starts = jnp.pad(starts, (0, 64 - (NW + 1)))
    xT = x.T
    p = _sc_spmv(xT, cols_p, rows_p, vals_p, starts, nnz_pad)
    return _tc_linear(p, W, bc_value, interior_flag)
